# baseline (speedup 1.0000x reference)
import jax
import jax.numpy as jnp
from jax.experimental import pallas as pl
from jax.experimental.pallas import tpu as pltpu

B, W, H, A, C = 16, 64, 64, 3, 80
WB = 64
WC = W // WB
BB = 2


def _loss_body(conf_ref, mask_ref, iou_ref, sw_ref, pxy_ref, pwh_ref,
               tb_ref, cls_ref, ts_ref,
               total_ref, noobj_ref, obj_ref, score_ref, xy_ref, wh_ref,
               acc_ref):
    b = pl.program_id(0)
    wc = pl.program_id(1)

    conf = conf_ref[...]
    mask = mask_ref[...]
    iou = iou_ref[...]
    sw = sw_ref[...]

    noobj_p = 0.25 * jnp.sum(jnp.where(mask == 0.0, conf * conf, 0.0))
    obj_p = 0.5 * jnp.sum(jnp.where(mask == 1.0, (conf - iou) ** 2, 0.0))

    shape = (BB, WB, A, 1, H)
    wi = jax.lax.broadcasted_iota(jnp.int32, shape, 1)
    a = jax.lax.broadcasted_iota(jnp.int32, shape, 2)
    h = jax.lax.broadcasted_iota(jnp.int32, shape, 4)
    n = (wc * WB + wi) * (H * A) + h * A + a
    fmx = ((n & 4095) >> 6).astype(jnp.float32)
    fmy = (n & 63).astype(jnp.float32)

    gtsw = jnp.where(sw > 0.0, sw, 0.0)

    x0 = pxy_ref[:, :, :, 0:1, :]
    x1 = pxy_ref[:, :, :, 1:2, :]
    w0 = pwh_ref[:, :, :, 0:1, :]
    w1 = pwh_ref[:, :, :, 1:2, :]
    t0 = tb_ref[:, :, :, 0:1, :]
    t1 = tb_ref[:, :, :, 1:2, :]
    t2 = tb_ref[:, :, :, 2:3, :]
    t3 = tb_ref[:, :, :, 3:4, :]

    def bce(x, t):
        return jnp.maximum(x, 0.0) - x * t + jnp.log1p(jnp.exp(-jnp.abs(x)))

    xy_p = 0.5 * jnp.sum(gtsw * (bce(x0 - fmx, t0 - fmx)
                                 + bce(x1 - fmy, t1 - fmy)))
    wh_p = 0.5 * jnp.sum(gtsw * ((w0 - t2) ** 2 + (w1 - t3) ** 2))

    d = cls_ref[...] - ts_ref[...]
    gt = (sw > 0.0).astype(jnp.float32)
    masked = jax.lax.dot_general(
        gt.reshape(BB * WB * A, 1, H), (d * d).reshape(BB * WB * A, H, C),
        dimension_numbers=(((2,), (1,)), ((0,), (0,))),
        preferred_element_type=jnp.float32)
    score_p = 0.5 * jnp.sum(masked)

    first = (b == 0) & (wc == 0)
    last = (b == B // BB - 1) & (wc == WC - 1)

    @pl.when(first)
    def _():
        acc_ref[0] = noobj_p
        acc_ref[1] = obj_p
        acc_ref[2] = xy_p
        acc_ref[3] = wh_p
        acc_ref[4] = score_p

    @pl.when(jnp.logical_not(first))
    def _():
        acc_ref[0] += noobj_p
        acc_ref[1] += obj_p
        acc_ref[2] += xy_p
        acc_ref[3] += wh_p
        acc_ref[4] += score_p

    @pl.when(last)
    def _():
        inv_b = 1.0 / B
        score_loss = acc_ref[4] * inv_b
        total_ref[0] = score_loss
        noobj_ref[0] = acc_ref[0] * (inv_b / 4.0)
        obj_ref[0] = acc_ref[1] * (inv_b / 4.0)
        score_ref[0] = score_loss / 4.0
        xy_ref[0] = acc_ref[2] * (inv_b / 4.0)
        wh_ref[0] = acc_ref[3] * (inv_b / 4.0)


def kernel(epoch, conf, pred_xy, pred_wh, cls_score, cls_out, obj_mask,
           true_bbox, true_score, pred_gt_iou, scale_weight):
    nat = lambda x: x.transpose(0, 1, 3, 4, 2)
    conf_n = nat(conf)
    mask_n = nat(obj_mask)
    iou_n = nat(pred_gt_iou)
    sw_n = nat(scale_weight)
    pxy_n = nat(pred_xy)
    pwh_n = nat(pred_wh)
    tb_n = nat(true_bbox)
    cls_n = cls_score.transpose(0, 1, 3, 2, 4)
    ts_n = true_score.transpose(0, 1, 3, 2, 4)

    def spec(k):
        return pl.BlockSpec((BB, WB, A, k, H), lambda b, wc: (b, wc, 0, 0, 0))

    big_spec = pl.BlockSpec((BB, WB, A, H, C), lambda b, wc: (b, wc, 0, 0, 0))
    scalar_out = pl.BlockSpec(memory_space=pltpu.SMEM)
    out_sds = jax.ShapeDtypeStruct((1,), jnp.float32)

    outs = pl.pallas_call(
        _loss_body,
        grid=(B // BB, WC),
        in_specs=[spec(1), spec(1), spec(1), spec(1), spec(2), spec(2),
                  spec(4), big_spec, big_spec],
        out_specs=[scalar_out] * 6,
        out_shape=[out_sds] * 6,
        scratch_shapes=[pltpu.SMEM((5,), jnp.float32)],
    )(conf_n, mask_n, iou_n, sw_n, pxy_n, pwh_n, tb_n, cls_n, ts_n)

    total, noobj_loss, obj_loss, score_loss, xy_loss, wh_loss = outs
    return (total[0], noobj_loss[0], obj_loss[0], score_loss[0],
            xy_loss[0], wh_loss[0])


# speedup vs baseline: 1.0473x; 1.0473x over previous
import jax
import jax.numpy as jnp
from jax.experimental import pallas as pl
from jax.experimental.pallas import tpu as pltpu

B, W, H, A, C = 16, 64, 64, 3, 80
WB = 64
WC = W // WB


def _loss_body(conf_ref, mask_ref, iou_ref, sw_ref, pxy_ref, pwh_ref,
               tb_ref, cls_ref, ts_ref,
               total_ref, noobj_ref, obj_ref, score_ref, xy_ref, wh_ref,
               acc_ref):
    b = pl.program_id(0)
    wc = pl.program_id(1)

    conf = conf_ref[0]
    mask = mask_ref[0]
    iou = iou_ref[0]
    sw = sw_ref[0]

    noobj_p = 0.25 * jnp.sum(jnp.where(mask == 0.0, conf * conf, 0.0))
    obj_p = 0.5 * jnp.sum(jnp.where(mask == 1.0, (conf - iou) ** 2, 0.0))

    shape = (WB, A, 1, H)
    wi = jax.lax.broadcasted_iota(jnp.int32, shape, 0)
    a = jax.lax.broadcasted_iota(jnp.int32, shape, 1)
    h = jax.lax.broadcasted_iota(jnp.int32, shape, 3)
    n = (wc * WB + wi) * (H * A) + h * A + a
    fmx = ((n & 4095) >> 6).astype(jnp.float32)
    fmy = (n & 63).astype(jnp.float32)

    gtsw = jnp.where(sw > 0.0, sw, 0.0)

    x0 = pxy_ref[0, :, :, 0:1, :]
    x1 = pxy_ref[0, :, :, 1:2, :]
    w0 = pwh_ref[0, :, :, 0:1, :]
    w1 = pwh_ref[0, :, :, 1:2, :]
    t0 = tb_ref[0, :, :, 0:1, :]
    t1 = tb_ref[0, :, :, 1:2, :]
    t2 = tb_ref[0, :, :, 2:3, :]
    t3 = tb_ref[0, :, :, 3:4, :]

    def bce(x, t):
        return jnp.maximum(x, 0.0) - x * t + jnp.log1p(jnp.exp(-jnp.abs(x)))

    xy_p = 0.5 * jnp.sum(gtsw * (bce(x0 - fmx, t0 - fmx)
                                 + bce(x1 - fmy, t1 - fmy)))
    wh_p = 0.5 * jnp.sum(gtsw * ((w0 - t2) ** 2 + (w1 - t3) ** 2))

    d = cls_ref[0] - ts_ref[0]
    gt = (sw > 0.0).astype(jnp.float32)
    masked = jax.lax.dot_general(
        gt.reshape(WB * A, 1, H), (d * d).reshape(WB * A, H, C),
        dimension_numbers=(((2,), (1,)), ((0,), (0,))),
        preferred_element_type=jnp.float32)
    score_p = 0.5 * jnp.sum(masked)

    first = (b == 0) & (wc == 0)
    last = (b == B - 1) & (wc == WC - 1)

    @pl.when(first)
    def _():
        acc_ref[0] = noobj_p
        acc_ref[1] = obj_p
        acc_ref[2] = xy_p
        acc_ref[3] = wh_p
        acc_ref[4] = score_p

    @pl.when(jnp.logical_not(first))
    def _():
        acc_ref[0] += noobj_p
        acc_ref[1] += obj_p
        acc_ref[2] += xy_p
        acc_ref[3] += wh_p
        acc_ref[4] += score_p

    @pl.when(last)
    def _():
        inv_b = 1.0 / B
        score_loss = acc_ref[4] * inv_b
        total_ref[0] = score_loss
        noobj_ref[0] = acc_ref[0] * (inv_b / 4.0)
        obj_ref[0] = acc_ref[1] * (inv_b / 4.0)
        score_ref[0] = score_loss / 4.0
        xy_ref[0] = acc_ref[2] * (inv_b / 4.0)
        wh_ref[0] = acc_ref[3] * (inv_b / 4.0)


def kernel(epoch, conf, pred_xy, pred_wh, cls_score, cls_out, obj_mask,
           true_bbox, true_score, pred_gt_iou, scale_weight):
    nat = lambda x: x.transpose(0, 1, 3, 4, 2)
    conf_n = nat(conf)
    mask_n = nat(obj_mask)
    iou_n = nat(pred_gt_iou)
    sw_n = nat(scale_weight)
    pxy_n = nat(pred_xy)
    pwh_n = nat(pred_wh)
    tb_n = nat(true_bbox)
    cls_n = cls_score.transpose(0, 1, 3, 2, 4)
    ts_n = true_score.transpose(0, 1, 3, 2, 4)

    def spec(k):
        return pl.BlockSpec((1, WB, A, k, H), lambda b, wc: (b, wc, 0, 0, 0))

    big_spec = pl.BlockSpec((1, WB, A, H, C), lambda b, wc: (b, wc, 0, 0, 0))
    scalar_out = pl.BlockSpec(memory_space=pltpu.SMEM)
    out_sds = jax.ShapeDtypeStruct((1,), jnp.float32)

    outs = pl.pallas_call(
        _loss_body,
        grid=(B, WC),
        in_specs=[spec(1), spec(1), spec(1), spec(1), spec(2), spec(2),
                  spec(4), big_spec, big_spec],
        out_specs=[scalar_out] * 6,
        out_shape=[out_sds] * 6,
        scratch_shapes=[pltpu.SMEM((5,), jnp.float32)],
    )(conf_n, mask_n, iou_n, sw_n, pxy_n, pwh_n, tb_n, cls_n, ts_n)

    total, noobj_loss, obj_loss, score_loss, xy_loss, wh_loss = outs
    return (total[0], noobj_loss[0], obj_loss[0], score_loss[0],
            xy_loss[0], wh_loss[0])
